# baseline (speedup 1.0000x reference)
import functools

import jax
import jax.numpy as jnp
from jax import lax
from jax.experimental import pallas as pl
from jax.experimental.pallas import tpu as pltpu
from jax.experimental.pallas import tpu_sc as plsc

N = 10000
NP = 10240
E = 160000
TILES = 16
CB = 128
NCHUNK = 79
EP = TILES * NCHUNK * CB
ROWS_PT = NP // TILES
CH = 128
BM = 512


def _fill_bf16(ref, rows, cols, val):
    v = jnp.full((32,), val, jnp.bfloat16)

    def body(r, _):
        for c in range(cols // 32):
            ref[r, pl.ds(c * 32, 32)] = v
        return 0

    lax.fori_loop(0, rows, body, 0)


def _make_agg(nc, with_deg):
    out_type = [jax.ShapeDtypeStruct((NP, CH), jnp.bfloat16) for _ in range(2 * nc)]
    if with_deg:
        out_type.append(jax.ShapeDtypeStruct((2 * TILES, NP), jnp.float32))

    scratch = [
        pltpu.VMEM((NCHUNK, CB), jnp.int32),
        pltpu.VMEM((NCHUNK, CB), jnp.int32),
        pltpu.VMEM((6, CB, CH), jnp.bfloat16),
        pltpu.VMEM((CB, CH), jnp.bfloat16),
        pltpu.VMEM_SHARED((NP, CH), jnp.bfloat16),
    ]
    if with_deg:
        scratch.append(pltpu.VMEM((NP,), jnp.float32))
    scratch.extend([pltpu.SemaphoreType.DMA] * 12)

    mesh = plsc.VectorSubcoreMesh(core_axis_name="c", subcore_axis_name="s")

    @functools.partial(pl.kernel, out_type=out_type, mesh=mesh,
                       scratch_types=scratch,
                       compiler_params=pltpu.CompilerParams(
                           needs_layout_passes=False,
                           use_tc_tiling_on_sc=False))
    def agg(src_hbm, dst_hbm, *rest):
        tables = rest[:2 * nc]
        outs = rest[2 * nc:4 * nc]
        k = 4 * nc
        if with_deg:
            deg_out = rest[k]; k += 1
        src_v = rest[k]; dst_v = rest[k + 1]
        rows_v = rest[k + 2]; zrow_v = rest[k + 3]
        acc = rest[k + 4]; k += 5
        if with_deg:
            hist_v = rest[k]; k += 1
        gsems = rest[k:k + 6]
        ssems = rest[k + 6:k + 12]

        core = lax.axis_index("c")
        sid = lax.axis_index("s")
        row0 = sid * ROWS_PT

        pltpu.sync_copy(src_hbm.at[sid], src_v)
        pltpu.sync_copy(dst_hbm.at[sid], dst_v)

        _fill_bf16(zrow_v, CB, CH, 0.0)
        if with_deg:
            zv = jnp.zeros((16,), jnp.float32)

            def zb(rr, _):
                hist_v[pl.ds(rr * 16, 16)] = zv
                return 0
            lax.fori_loop(0, NP // 16, zb, 0)

            ones16 = jnp.ones((16,), jnp.float32)

            def db(j, _):
                for c16 in range(CB // 16):
                    idx = dst_v[j, pl.ds(c16 * 16, 16)]
                    plsc.addupdate_scatter(hist_v, [idx], ones16)
                return 0
            half = NCHUNK // 2
            lax.fori_loop(core * half, half + core * (NCHUNK - half),
                          db, 0)
            pltpu.sync_copy(hist_v, deg_out.at[core * TILES + sid])

        for i in range(nc):
            for b in range(ROWS_PT // CB):
                pltpu.sync_copy(zrow_v, acc.at[pl.ds(row0 + b * CB, CB)])
            plsc.subcore_barrier()

            for c in range(2):
                @pl.when(core == c)
                def _(c=c, i=i):
                    tab = tables[c * nc + i]

                    def g_issue(j, p):
                        pltpu.async_copy(tab.at[src_v.at[j]], rows_v.at[p],
                                         gsems[p])

                    def g_wait(j, p):
                        pltpu.make_async_copy(tab.at[src_v.at[j]],
                                              rows_v.at[p], gsems[p]).wait()

                    def s_issue(j, p):
                        pltpu.async_copy(rows_v.at[p], acc.at[dst_v.at[j]],
                                         ssems[p], add=True)

                    def s_wait(j, p):
                        pltpu.make_async_copy(rows_v.at[p],
                                              acc.at[dst_v.at[j]],
                                              ssems[p]).wait()

                    for p in range(6):
                        g_issue(p, p)

                    def edge_body(t6, _):
                        j = 6 * t6
                        for p in range(6):
                            m = j + p
                            g_wait(m, p); s_issue(m, p)
                            pp = (p + 3) % 6

                            @pl.when(m >= 3)
                            def _(m=m, pp=pp):
                                s_wait(m - 3, pp)

                                @pl.when(m + 3 < NCHUNK)
                                def _():
                                    g_issue(m + 3, pp)
                        return 0

                    lax.fori_loop(0, NCHUNK // 6, edge_body, 0)
                    for p in range(NCHUNK % 6):
                        m = NCHUNK - NCHUNK % 6 + p
                        g_wait(m, m % 6); s_issue(m, m % 6)
                        s_wait(m - 3, (m - 3) % 6)
                        if m + 3 < NCHUNK:
                            g_issue(m + 3, (m + 3) % 6)
                    for m in range(NCHUNK - 3, NCHUNK):
                        s_wait(m, m % 6)
            plsc.subcore_barrier()

            for c in range(2):
                @pl.when(core == c)
                def _(c=c, i=i):
                    pltpu.sync_copy(acc.at[pl.ds(row0, ROWS_PT)],
                                    outs[c * nc + i].at[pl.ds(row0, ROWS_PT)])

    return agg


_agg1 = _make_agg(nc=1, with_deg=True)
_agg2 = _make_agg(nc=2, with_deg=False)


def _tc1_body(x_ref, a0_ref, a1_ref, d_ref, ws_ref, wn_ref,
              b_ref, *o_refs):
    deg = jnp.sum(d_ref[...], axis=0).reshape(BM, 1)
    r = 1.0 / jnp.maximum(deg, 1.0)
    agg = (jnp.concatenate([a0_ref[...], a1_ref[...]], axis=1)
           .astype(jnp.float32) * r).astype(jnp.bfloat16)
    h = jnp.dot(x_ref[...], ws_ref[...], preferred_element_type=jnp.float32)
    h = h + jnp.dot(agg, wn_ref[...], preferred_element_type=jnp.float32)
    h = jnp.maximum(h + b_ref[...], 0.0).astype(jnp.bfloat16)
    for c in range(4):
        o_refs[c][...] = h[:, c * CH:(c + 1) * CH]


_tc1 = pl.pallas_call(
    _tc1_body,
    grid=(NP // BM,),
    in_specs=[
        pl.BlockSpec((BM, 256), lambda m: (m, 0)),
        pl.BlockSpec((BM, CH), lambda m: (m, 0)),
        pl.BlockSpec((BM, CH), lambda m: (m, 0)),
        pl.BlockSpec((2 * TILES, BM), lambda m: (0, m)),
        pl.BlockSpec((256, 512), lambda m: (0, 0)),
        pl.BlockSpec((256, 512), lambda m: (0, 0)),
        pl.BlockSpec((1, 512), lambda m: (0, 0)),
    ],
    out_specs=[pl.BlockSpec((BM, CH), lambda m: (m, 0)) for _ in range(4)],
    out_shape=[jax.ShapeDtypeStruct((NP, CH), jnp.bfloat16) for _ in range(4)],
)


def _tc2_body(h0_ref, h1_ref, h2_ref, h3_ref,
              g0_ref, g1_ref, g2_ref, g3_ref, d_ref, ws_ref, wn_ref,
              b1_ref, rw0_ref, rb0_ref, rw1_ref, rb1_ref, rw2_ref, rb2_ref,
              o_ref):
    deg = jnp.sum(d_ref[...], axis=0).reshape(BM, 1)
    r = 1.0 / jnp.maximum(deg, 1.0)
    h = jnp.concatenate(
        [h0_ref[...], h1_ref[...], h2_ref[...], h3_ref[...]], axis=1)
    neigh = (jnp.concatenate(
        [g0_ref[...], g1_ref[...], g2_ref[...], g3_ref[...]],
        axis=1).astype(jnp.float32) * r).astype(jnp.bfloat16)
    h2 = jnp.dot(h, ws_ref[...], preferred_element_type=jnp.float32)
    h2 = h2 + jnp.dot(neigh, wn_ref[...], preferred_element_type=jnp.float32)
    h2 = jnp.maximum(h2 + b1_ref[...], 0.0).astype(jnp.bfloat16)
    t = jnp.maximum(
        jnp.dot(h2, rw0_ref[...], preferred_element_type=jnp.float32)
        + rb0_ref[...], 0.0).astype(jnp.bfloat16)
    t = jnp.maximum(
        jnp.dot(t, rw1_ref[...], preferred_element_type=jnp.float32)
        + rb1_ref[...], 0.0).astype(jnp.bfloat16)
    o_ref[...] = (jnp.dot(t, rw2_ref[...], preferred_element_type=jnp.float32)
                  + rb2_ref[...])


_tc2 = pl.pallas_call(
    _tc2_body,
    grid=(NP // BM,),
    in_specs=(
        [pl.BlockSpec((BM, CH), lambda m: (m, 0)) for _ in range(8)]
        + [
            pl.BlockSpec((2 * TILES, BM), lambda m: (0, m)),
            pl.BlockSpec((512, 512), lambda m: (0, 0)),
            pl.BlockSpec((512, 512), lambda m: (0, 0)),
            pl.BlockSpec((1, 512), lambda m: (0, 0)),
            pl.BlockSpec((512, 512), lambda m: (0, 0)),
            pl.BlockSpec((1, 512), lambda m: (0, 0)),
            pl.BlockSpec((512, 256), lambda m: (0, 0)),
            pl.BlockSpec((1, 256), lambda m: (0, 0)),
            pl.BlockSpec((256, 1), lambda m: (0, 0)),
            pl.BlockSpec((1, 1), lambda m: (0, 0)),
        ]
    ),
    out_specs=pl.BlockSpec((BM, 1), lambda m: (m, 0)),
    out_shape=jax.ShapeDtypeStruct((NP, 1), jnp.float32),
)


def kernel(x, edge_index, W_self0, W_neigh0, b0, W_self1, W_neigh1, b1,
           R_W0, R_b0, R_W1, R_b1, R_W2, R_b2):
    src = edge_index[0]
    dst = edge_index[1]
    padv = jnp.full((EP - E,), NP - 1, jnp.int32)
    src_p = jnp.concatenate([src, padv]).reshape(TILES, NCHUNK, CB)
    dst_p = jnp.concatenate([dst, padv]).reshape(TILES, NCHUNK, CB)
    xp = jnp.pad(x, ((0, NP - N), (0, 0)))

    bf = jnp.bfloat16
    xb = xp.astype(bf)
    xs = [lax.slice(xb, (0, c * CH), (NP, (c + 1) * CH)) for c in range(2)]
    a0, a1, degp = _agg1(src_p, dst_p, *xs)
    hs = _tc1(xb, a0, a1, degp, W_self0.astype(bf), W_neigh0.astype(bf),
              b0.reshape(1, -1))
    gs = _agg2(src_p, dst_p, *hs)
    out = _tc2(*hs, *gs, degp, W_self1.astype(bf), W_neigh1.astype(bf),
               b1.reshape(1, -1), R_W0.astype(bf), R_b0.reshape(1, -1),
               R_W1.astype(bf), R_b1.reshape(1, -1), R_W2.astype(bf),
               R_b2.reshape(1, -1))
    return out[:N]


# speedup vs baseline: 1.0260x; 1.0260x over previous
import functools

import jax
import jax.numpy as jnp
from jax import lax
from jax.experimental import pallas as pl
from jax.experimental.pallas import tpu as pltpu
from jax.experimental.pallas import tpu_sc as plsc

N = 10000
NP = 10240
E = 160000
TILES = 16
CB = 128
NCHUNK = 79
EP = TILES * NCHUNK * CB
ROWS_PT = NP // TILES
CH = 128
BM = 512


def _fill_bf16(ref, rows, cols, val):
    v = jnp.full((32,), val, jnp.bfloat16)

    def body(r, _):
        for c in range(cols // 32):
            ref[r, pl.ds(c * 32, 32)] = v
        return 0

    lax.fori_loop(0, rows, body, 0)


def _make_agg(nc, with_deg):
    out_type = [jax.ShapeDtypeStruct((NP, CH), jnp.bfloat16) for _ in range(2 * nc)]
    if with_deg:
        out_type.append(jax.ShapeDtypeStruct((2 * TILES, NP), jnp.float32))

    scratch = [
        pltpu.VMEM((NCHUNK, CB), jnp.int32),
        pltpu.VMEM((NCHUNK, CB), jnp.int32),
        pltpu.VMEM((6, CB, CH), jnp.bfloat16),
        pltpu.VMEM((CB, CH), jnp.bfloat16),
        pltpu.VMEM_SHARED((NP, CH), jnp.bfloat16),
    ]
    if with_deg:
        scratch.append(pltpu.VMEM((NP,), jnp.float32))
    scratch.extend([pltpu.SemaphoreType.DMA] * 12)

    mesh = plsc.VectorSubcoreMesh(core_axis_name="c", subcore_axis_name="s")

    @functools.partial(pl.kernel, out_type=out_type, mesh=mesh,
                       scratch_types=scratch,
                       compiler_params=pltpu.CompilerParams(
                           needs_layout_passes=False,
                           use_tc_tiling_on_sc=False))
    def agg(src_hbm, dst_hbm, *rest):
        tables = rest[:2 * nc]
        outs = rest[2 * nc:4 * nc]
        k = 4 * nc
        if with_deg:
            deg_out = rest[k]; k += 1
        src_v = rest[k]; dst_v = rest[k + 1]
        rows_v = rest[k + 2]; zrow_v = rest[k + 3]
        acc = rest[k + 4]; k += 5
        if with_deg:
            hist_v = rest[k]; k += 1
        gsems = rest[k:k + 6]
        ssems = rest[k + 6:k + 12]

        core = lax.axis_index("c")
        sid = lax.axis_index("s")
        row0 = sid * ROWS_PT

        pltpu.sync_copy(src_hbm.at[sid], src_v)
        pltpu.sync_copy(dst_hbm.at[sid], dst_v)

        _fill_bf16(zrow_v, CB, CH, 0.0)
        if with_deg:
            zv = jnp.zeros((16,), jnp.float32)

            def zb(rr, _):
                hist_v[pl.ds(rr * 16, 16)] = zv
                return 0
            lax.fori_loop(0, NP // 16, zb, 0)

            ones16 = jnp.ones((16,), jnp.float32)

            def db(j, _):
                for c16 in range(CB // 16):
                    idx = dst_v[j, pl.ds(c16 * 16, 16)]
                    plsc.addupdate_scatter(hist_v, [idx], ones16)
                return 0
            half = NCHUNK // 2
            lax.fori_loop(core * half, half + core * (NCHUNK - half), db, 0)
            pltpu.sync_copy(hist_v, deg_out.at[core * TILES + sid])

        for i in range(nc):
            for b in range(ROWS_PT // CB):
                pltpu.sync_copy(zrow_v, acc.at[pl.ds(row0 + b * CB, CB)])
            plsc.subcore_barrier()

            for c in range(2):
                @pl.when(core == c)
                def _(c=c, i=i):
                    tab = tables[c * nc + i]

                    def g_issue(j, p):
                        pltpu.async_copy(tab.at[src_v.at[j]], rows_v.at[p],
                                         gsems[p])

                    def g_wait(j, p):
                        pltpu.make_async_copy(tab.at[src_v.at[j]],
                                              rows_v.at[p], gsems[p]).wait()

                    def s_issue(j, p):
                        pltpu.async_copy(rows_v.at[p], acc.at[dst_v.at[j]],
                                         ssems[p], add=True)

                    def s_wait(j, p):
                        pltpu.make_async_copy(rows_v.at[p],
                                              acc.at[dst_v.at[j]],
                                              ssems[p]).wait()

                    for p in range(6):
                        g_issue(p, p)

                    def edge_body(t6, _):
                        j = 6 * t6
                        for p in range(6):
                            m = j + p
                            g_wait(m, p); s_issue(m, p)
                            pp = (p + 3) % 6

                            @pl.when(m >= 3)
                            def _(m=m, pp=pp):
                                s_wait(m - 3, pp)

                                @pl.when(m + 3 < NCHUNK)
                                def _():
                                    g_issue(m + 3, pp)
                        return 0

                    lax.fori_loop(0, NCHUNK // 6, edge_body, 0)
                    for p in range(NCHUNK % 6):
                        m = NCHUNK - NCHUNK % 6 + p
                        g_wait(m, m % 6); s_issue(m, m % 6)
                        s_wait(m - 3, (m - 3) % 6)
                        if m + 3 < NCHUNK:
                            g_issue(m + 3, (m + 3) % 6)
                    for m in range(NCHUNK - 3, NCHUNK):
                        s_wait(m, m % 6)
            plsc.subcore_barrier()

            for c in range(2):
                @pl.when(core == c)
                def _(c=c, i=i):
                    pltpu.sync_copy(acc.at[pl.ds(row0, ROWS_PT)],
                                    outs[c * nc + i].at[pl.ds(row0, ROWS_PT)])

    return agg


_agg1 = _make_agg(nc=1, with_deg=True)
_agg2 = _make_agg(nc=2, with_deg=False)


def _tc1_body(x_ref, a0_ref, a1_ref, d_ref, ws_ref, wn_ref,
              b_ref, *o_refs):
    deg = jnp.sum(d_ref[...], axis=0).reshape(BM, 1)
    r = 1.0 / jnp.maximum(deg, 1.0)
    agg = jnp.concatenate(
        [a0_ref[...], a1_ref[...]],
        axis=1).astype(jnp.float32) * r
    h = jnp.dot(x_ref[...], ws_ref[...], preferred_element_type=jnp.float32)
    h = h + jnp.dot(agg, wn_ref[...], preferred_element_type=jnp.float32)
    h = jnp.maximum(h + b_ref[...], 0.0).astype(jnp.bfloat16)
    for c in range(4):
        o_refs[c][...] = h[:, c * CH:(c + 1) * CH]


_tc1 = pl.pallas_call(
    _tc1_body,
    grid=(NP // BM,),
    in_specs=[
        pl.BlockSpec((BM, 256), lambda m: (m, 0)),
        pl.BlockSpec((BM, CH), lambda m: (m, 0)),
        pl.BlockSpec((BM, CH), lambda m: (m, 0)),
        pl.BlockSpec((2 * TILES, BM), lambda m: (0, m)),
        pl.BlockSpec((256, 512), lambda m: (0, 0)),
        pl.BlockSpec((256, 512), lambda m: (0, 0)),
        pl.BlockSpec((1, 512), lambda m: (0, 0)),
    ],
    out_specs=[pl.BlockSpec((BM, CH), lambda m: (m, 0)) for _ in range(4)],
    out_shape=[jax.ShapeDtypeStruct((NP, CH), jnp.bfloat16) for _ in range(4)],
)


def _tc2_body(h0_ref, h1_ref, h2_ref, h3_ref,
              g0_ref, g1_ref, g2_ref, g3_ref, d_ref, ws_ref, wn_ref,
              b1_ref, rw0_ref, rb0_ref, rw1_ref, rb1_ref, rw2_ref, rb2_ref,
              o_ref):
    deg = jnp.sum(d_ref[...], axis=0).reshape(BM, 1)
    r = 1.0 / jnp.maximum(deg, 1.0)
    h = jnp.concatenate(
        [h0_ref[...], h1_ref[...], h2_ref[...], h3_ref[...]],
        axis=1).astype(jnp.float32)
    neigh = jnp.concatenate(
        [g0_ref[...], g1_ref[...], g2_ref[...], g3_ref[...]],
        axis=1).astype(jnp.float32) * r
    h2 = jnp.dot(h, ws_ref[...], preferred_element_type=jnp.float32)
    h2 = h2 + jnp.dot(neigh, wn_ref[...], preferred_element_type=jnp.float32)
    h2 = jnp.maximum(h2 + b1_ref[...], 0.0)
    t = jnp.maximum(
        jnp.dot(h2, rw0_ref[...], preferred_element_type=jnp.float32)
        + rb0_ref[...], 0.0)
    t = jnp.maximum(
        jnp.dot(t, rw1_ref[...], preferred_element_type=jnp.float32)
        + rb1_ref[...], 0.0)
    o_ref[...] = (jnp.dot(t, rw2_ref[...], preferred_element_type=jnp.float32)
                  + rb2_ref[...])


_tc2 = pl.pallas_call(
    _tc2_body,
    grid=(NP // BM,),
    in_specs=(
        [pl.BlockSpec((BM, CH), lambda m: (m, 0)) for _ in range(8)]
        + [
            pl.BlockSpec((2 * TILES, BM), lambda m: (0, m)),
            pl.BlockSpec((512, 512), lambda m: (0, 0)),
            pl.BlockSpec((512, 512), lambda m: (0, 0)),
            pl.BlockSpec((1, 512), lambda m: (0, 0)),
            pl.BlockSpec((512, 512), lambda m: (0, 0)),
            pl.BlockSpec((1, 512), lambda m: (0, 0)),
            pl.BlockSpec((512, 256), lambda m: (0, 0)),
            pl.BlockSpec((1, 256), lambda m: (0, 0)),
            pl.BlockSpec((256, 1), lambda m: (0, 0)),
            pl.BlockSpec((1, 1), lambda m: (0, 0)),
        ]
    ),
    out_specs=pl.BlockSpec((BM, 1), lambda m: (m, 0)),
    out_shape=jax.ShapeDtypeStruct((NP, 1), jnp.float32),
)


def kernel(x, edge_index, W_self0, W_neigh0, b0, W_self1, W_neigh1, b1,
           R_W0, R_b0, R_W1, R_b1, R_W2, R_b2):
    src = edge_index[0]
    dst = edge_index[1]
    padv = jnp.full((EP - E,), NP - 1, jnp.int32)
    src_p = jnp.concatenate([src, padv]).reshape(TILES, NCHUNK, CB)
    dst_p = jnp.concatenate([dst, padv]).reshape(TILES, NCHUNK, CB)
    xp = jnp.pad(x, ((0, NP - N), (0, 0)))

    xb = xp.astype(jnp.bfloat16)
    xs = [lax.slice(xb, (0, c * CH), (NP, (c + 1) * CH)) for c in range(2)]
    a0, a1, degp = _agg1(src_p, dst_p, *xs)
    hs = _tc1(xp, a0, a1, degp, W_self0, W_neigh0,
              b0.reshape(1, -1))
    gs = _agg2(src_p, dst_p, *hs)
    out = _tc2(*hs, *gs, degp, W_self1, W_neigh1,
               b1.reshape(1, -1), R_W0, R_b0.reshape(1, -1),
               R_W1, R_b1.reshape(1, -1), R_W2, R_b2.reshape(1, -1))
    return out[:N]
